# Initial kernel scaffold; baseline (speedup 1.0000x reference)
import jax
import jax.numpy as jnp
from jax.experimental import pallas as pl


def kernel(x, Wg, bg, W1, b1, W2, b2):
    raise NotImplementedError("write your pallas kernel here")


# speedup vs baseline: 1.8573x; 1.8573x over previous
import jax
import jax.numpy as jnp
from jax.experimental import pallas as pl
from jax.experimental.pallas import tpu as pltpu

DIM_IN = 768
NUM_CLASSES = 256
NUM_EXPERTS = 8
HIDDEN = 256
TOKENS = 4096
TB = 512
NTB = TOKENS // TB


def _moe_block(x_ref, Wg_ref, bg_ref, W1_ref, b1_ref, W2_ref, b2_ref,
               out_ref, lbl_ref, psum_ref):
    tb = pl.program_id(0)
    x = x_ref[...]

    logits = jnp.dot(x, Wg_ref[...], preferred_element_type=jnp.float32)
    logits = logits + bg_ref[...]
    m = jnp.max(logits, axis=-1, keepdims=True)
    ex = jnp.exp(logits - m)
    probs = ex / jnp.sum(ex, axis=-1, keepdims=True)

    iota = jax.lax.broadcasted_iota(jnp.int32, probs.shape, 1)
    i1 = jnp.argmax(probs, axis=-1)
    oh1 = iota == i1[:, None]
    m1 = jnp.max(probs, axis=-1)
    probs_m = jnp.where(oh1, -1.0, probs)
    i2 = jnp.argmax(probs_m, axis=-1)
    oh2 = iota == i2[:, None]
    m2 = jnp.max(probs_m, axis=-1)
    denom = m1 + m2
    w = (oh1 * (m1 / denom)[:, None] + oh2 * (m2 / denom)[:, None])
    w = w.astype(jnp.float32)

    @pl.when(tb == 0)
    def _init():
        psum_ref[...] = jnp.zeros_like(psum_ref)

    psum_ref[...] += jnp.sum(probs, axis=0)[None, :]

    acc = jnp.zeros((TB, NUM_CLASSES), jnp.float32)
    for ei in range(NUM_EXPERTS):
        h = jnp.dot(x, W1_ref[ei], preferred_element_type=jnp.float32)
        h = jnp.maximum(h + b1_ref[ei], 0.0)
        y = jnp.dot(h, W2_ref[ei], preferred_element_type=jnp.float32)
        y = y + b2_ref[ei]
        acc = acc + w[:, ei][:, None] * y
    out_ref[...] = acc

    @pl.when(tb == NTB - 1)
    def _fin():
        mean = psum_ref[...] / TOKENS
        lbl_ref[...] = (NUM_EXPERTS * jnp.sum(mean * mean)).reshape(1, 1)


def kernel(x, Wg, bg, W1, b1, W2, b2):
    bg2 = bg.reshape(1, NUM_EXPERTS)
    out, lbl = pl.pallas_call(
        _moe_block,
        grid=(NTB,),
        in_specs=[
            pl.BlockSpec((TB, DIM_IN), lambda i: (i, 0)),
            pl.BlockSpec((DIM_IN, NUM_EXPERTS), lambda i: (0, 0)),
            pl.BlockSpec((1, NUM_EXPERTS), lambda i: (0, 0)),
            pl.BlockSpec((NUM_EXPERTS, DIM_IN, HIDDEN), lambda i: (0, 0, 0)),
            pl.BlockSpec((NUM_EXPERTS, HIDDEN), lambda i: (0, 0)),
            pl.BlockSpec((NUM_EXPERTS, HIDDEN, NUM_CLASSES), lambda i: (0, 0, 0)),
            pl.BlockSpec((NUM_EXPERTS, NUM_CLASSES), lambda i: (0, 0)),
        ],
        out_specs=[
            pl.BlockSpec((TB, NUM_CLASSES), lambda i: (i, 0)),
            pl.BlockSpec((1, 1), lambda i: (0, 0)),
        ],
        out_shape=[
            jax.ShapeDtypeStruct((TOKENS, NUM_CLASSES), jnp.float32),
            jax.ShapeDtypeStruct((1, 1), jnp.float32),
        ],
        scratch_shapes=[pltpu.VMEM((1, NUM_EXPERTS), jnp.float32)],
        compiler_params=pltpu.CompilerParams(
            dimension_semantics=("arbitrary",),
        ),
    )(x, Wg, bg2, W1, b1, W2, b2)
    return out, lbl[0, 0]
